# baseline (speedup 1.0000x reference)
import functools

import jax
import jax.numpy as jnp
import numpy as np
from jax import lax
from jax.experimental import pallas as pl
from jax.experimental.pallas import tpu as pltpu
from jax.experimental.pallas import tpu_sc as plsc

GRID = 32
STEP = 1.0 / GRID
NUM_STEPS = 4
HEADS = 4
HID = 64
B = 64
N = B * GRID
F1 = HEADS * HID


_HI = jax.lax.Precision.HIGHEST


def _dotx(a, b):
    return jnp.dot(a, b, preferred_element_type=jnp.float32, precision=_HI)


def _b32(x):
    return x.astype(jnp.bfloat16).astype(jnp.float32)

def _roll(h, s):
    if s == 0:
        return h
    return jnp.concatenate([h[:, -s:, :], h[:, :-s, :]], axis=1)


def _elu(x):
    return jnp.where(x > 0, x, jnp.exp(jnp.minimum(x, 0.0)) - 1.0)


def _leaky(x):
    return jnp.where(x >= 0, x, 0.2 * x)


def _src_3d(ts):
    xi = -(ts + 2 * ts + 2 * ts + ts) / 6.0
    gio = lax.broadcasted_iota(jnp.int32, (B, GRID, 1), 1).astype(jnp.float32)
    x_n = (gio + 0.5) * STEP + xi
    x_n = x_n - jnp.floor(x_n)
    kio = lax.broadcasted_iota(jnp.int32, (B, GRID, GRID), 2).astype(jnp.float32)
    cmp = ((kio + 0.5) * STEP <= x_n).astype(jnp.int32)
    indx = jnp.sum(cmp, axis=2, keepdims=True) - 1
    return indx & 31, (indx + 1) & 31, xi


def _idx_body(ts_ref, f1_ref, f2_ref):
    src1, src2, _ = _src_3d(ts_ref[...])
    bio = lax.broadcasted_iota(jnp.int32, (B, GRID, 1), 0)
    f1_ref[...] = src1 + bio * GRID
    f2_ref[...] = src2 + bio * GRID


def _tc1_body(ts_ref, xg_ref, *refs):
    conv = [(refs[2 * i][...], refs[2 * i + 1][...]) for i in range(6)]
    wl, bl, wr, br = (refs[12][...], refs[13][...], refs[14][...], refs[15][...])
    xl_ref, xr_ref = refs[16], refs[17]
    ts = ts_ref[...]
    xi = -(ts + 2 * ts + 2 * ts + ts) / 6.0
    xi_ch = jnp.broadcast_to(xi * np.float32(GRID), (B, GRID, 1))
    h = jnp.concatenate([xg_ref[...], xi_ch], axis=2)
    for (W5, b5) in conv:
        ci5, co = W5.shape
        X5 = jnp.concatenate([_roll(h, s) for s in (2, 1, 0, -1, -2)], axis=2)
        hh = jnp.dot(X5.reshape(N, ci5), W5, preferred_element_type=jnp.float32) + b5
        h = _elu(hh).reshape(B, GRID, co)
    xc2 = h.reshape(N, 128)
    xl_ref[...] = jnp.dot(xc2, wl, preferred_element_type=jnp.float32) + bl
    xr_ref[...] = jnp.dot(xc2, wr, preferred_element_type=jnp.float32) + br


def _att_lin_body(a1_ref, a2_ref, xl_ref, xr_ref, am_ref, hmt_ref, bs_ref,
                  wl2_ref, bl2_ref, wr2_ref, br2_ref, xl2_ref, xr2_ref):
    A1, A2 = a1_ref[...], a2_ref[...]
    xl, xr = xl_ref[...], xr_ref[...]
    am, hmt, bs = am_ref[...], hmt_ref[...], bs_ref[...]
    l1 = _dotx(_leaky(A1 + xr), am)
    l2 = _dotx(_leaky(A2 + xr), am)
    ls = _dotx(_leaky(xl + xr), am)
    m = jnp.maximum(jnp.maximum(l1, l2), ls)
    a1 = jnp.exp(l1 - m)
    a2 = jnp.exp(l2 - m)
    as_ = jnp.exp(ls - m)
    den = a1 + a2 + as_
    wb = lambda a: _dotx(a / den, hmt)
    xc2 = wb(a1) * A1 + wb(a2) * A2 + wb(as_) * xl + bs
    xl2_ref[...] = jnp.dot(xc2, wl2_ref[...], preferred_element_type=jnp.float32) + bl2_ref[...]
    xr2_ref[...] = jnp.dot(xc2, wr2_ref[...], preferred_element_type=jnp.float32) + br2_ref[...]


def _fin_body(ts_ref, a1_ref, a2_ref, xl_ref, xr_ref, am_ref, hmt_ref,
              bs_ref, w2_ref, xg_ref, xlane_ref,
              newxg_ref, newxlane_ref, pred_ref):
    A1, A2 = a1_ref[...], a2_ref[...]
    xl, xr = xl_ref[...], xr_ref[...]
    am, hmt, bs = am_ref[...], hmt_ref[...], bs_ref[...]
    l1 = _dotx(_leaky(A1 + xr), am)
    l2 = _dotx(_leaky(A2 + xr), am)
    ls = _dotx(_leaky(xl + xr), am)
    m = jnp.maximum(jnp.maximum(l1, l2), ls)
    a1 = jnp.exp(l1 - m)
    a2 = jnp.exp(l2 - m)
    as_ = jnp.exp(ls - m)
    den = a1 + a2 + as_
    wb = lambda a: _dotx(a / den, hmt)
    xc2 = _elu(wb(a1) * A1 + wb(a2) * A2 + wb(as_) * xl + bs)

    src1, src2, _ = _src_3d(ts_ref[...])
    sio = lax.broadcasted_iota(jnp.int32, (B, GRID, GRID), 2)
    oh1 = (src1 == sio).astype(jnp.float32)
    oh2 = (src2 == sio).astype(jnp.float32)
    ohsum = oh1 + oh2
    deg = jnp.sum(ohsum, axis=1, keepdims=True)
    gio_i = lax.broadcasted_iota(jnp.int32, (B, GRID, GRID), 1)
    ident = (gio_i == sio).astype(jnp.float32)

    xc3 = xc2.reshape(B, GRID, F1)
    v3 = jnp.sum(_b32(xc3) * _b32(w2_ref[...]), axis=2, keepdims=True)
    Sv = jnp.sum(ohsum * v3, axis=1, keepdims=True)
    q = (Sv - 1.0) / deg
    q1 = jnp.sum(oh1 * q, axis=2, keepdims=True)
    q2 = jnp.sum(oh2 * q, axis=2, keepdims=True)
    xlane = xlane_ref[...]
    xs1 = jnp.sum(oh1 * xlane, axis=2, keepdims=True)
    xs2 = jnp.sum(oh2 * xlane, axis=2, keepdims=True)
    xg3 = xs1 * (v3 - q1) + xs2 * (v3 - q2)
    newxg_ref[...] = xg3
    newxlane_ref[...] = jnp.sum(ident * xg3, axis=1, keepdims=True)
    pred_ref[...] = xg3.reshape(N, 1)


_info = None


def _sc_gather(table, idx1, idx2):
    NW = 32
    b_per_w = N // NW
    mesh = plsc.VectorSubcoreMesh(core_axis_name="c", subcore_axis_name="s")

    @functools.partial(
        pl.kernel, mesh=mesh,
        out_type=(jax.ShapeDtypeStruct((N, F1), jnp.float32),
                  jax.ShapeDtypeStruct((N, F1), jnp.float32)),
        scratch_types=[
            pltpu.VMEM((b_per_w,), jnp.int32),
            pltpu.VMEM((b_per_w,), jnp.int32),
            pltpu.VMEM((b_per_w, F1), jnp.float32),
            pltpu.VMEM((b_per_w, F1), jnp.float32),
            pltpu.SemaphoreType.DMA,
            pltpu.SemaphoreType.DMA,
        ],
    )
    def k(table_hbm, i1_hbm, i2_hbm, o1_hbm, o2_hbm,
          i1_v, i2_v, r1_v, r2_v, sem1, sem2):
        wid = lax.axis_index("s") * 2 + lax.axis_index("c")
        base = wid * b_per_w
        pltpu.sync_copy(i1_hbm.at[pl.ds(base, b_per_w)], i1_v)
        pltpu.sync_copy(i2_hbm.at[pl.ds(base, b_per_w)], i2_v)
        c1 = pltpu.async_copy(table_hbm.at[i1_v], r1_v, sem1)
        c2 = pltpu.async_copy(table_hbm.at[i2_v], r2_v, sem2)
        c1.wait()
        c2.wait()
        pltpu.sync_copy(r1_v, o1_hbm.at[pl.ds(base, b_per_w)])
        pltpu.sync_copy(r2_v, o2_hbm.at[pl.ds(base, b_per_w)])

    return k(table, idx1, idx2)


def _tc_call(body, args, out_shapes):
    return pl.pallas_call(
        body,
        out_shape=out_shapes,
    )(*args)


@jax.jit
def kernel(x, cur_time, time_step, conv_params, gat1, gat2, lin_W, lin_b):
    del cur_time, lin_b
    ts3 = time_step.astype(jnp.float32).reshape(B, 1, 1)
    xg3 = x.reshape(B, GRID, 1)
    xlane = x.reshape(B, 1, GRID)

    flat_conv = []
    for (W, b) in conv_params:
        co, ci, _ = W.shape
        flat_conv.append(jnp.transpose(W, (2, 1, 0)).reshape(5 * ci, co))
        flat_conv.append(b.reshape(1, co))

    def gat_args(g):
        Wl, bl, Wr, br, att, bias = g
        attf = att.reshape(F1)
        hsel = (np.arange(F1)[:, None] // HID) == np.arange(HEADS)[None, :]
        am = attf[:, None] * jnp.asarray(hsel, dtype=jnp.float32)
        return [Wl, bl.reshape(1, F1), Wr, br.reshape(1, F1), am,
                bias.reshape(1, F1)]

    g1 = gat_args(gat1)
    g2 = gat_args(gat2)
    hmt = jnp.asarray((np.arange(F1)[None, :] // HID)
                      == np.arange(HEADS)[:, None], dtype=jnp.float32)
    w2 = lin_W[F1:, 0].reshape(1, 1, F1)

    S = jax.ShapeDtypeStruct
    f1g, f2g = _tc_call(_idx_body, [ts3],
                        (S((B, GRID, 1), jnp.int32), S((B, GRID, 1), jnp.int32)))
    idx1 = f1g.reshape(N)
    idx2 = f2g.reshape(N)

    preds = []
    for _ in range(NUM_STEPS):
        xl1, xr1 = _tc_call(
            _tc1_body,
            [ts3, xg3] + flat_conv + [g1[0], g1[1], g1[2], g1[3]],
            (S((N, F1), jnp.float32), S((N, F1), jnp.float32)))
        A1, A2 = _sc_gather(xl1, idx1, idx2)
        xl2, xr2 = _tc_call(
            _att_lin_body,
            [A1, A2, xl1, xr1, g1[4], hmt, g1[5], g2[0], g2[1], g2[2], g2[3]],
            (S((N, F1), jnp.float32), S((N, F1), jnp.float32)))
        B1, B2 = _sc_gather(xl2, idx1, idx2)
        xg3, xlane, pred = _tc_call(
            _fin_body,
            [ts3, B1, B2, xl2, xr2, g2[4], hmt, g2[5], w2, xg3, xlane],
            (S((B, GRID, 1), jnp.float32), S((B, 1, GRID), jnp.float32),
             S((N, 1), jnp.float32)))
        preds.append(pred)
    return jnp.concatenate(preds, axis=1)


# speedup vs baseline: 1.0009x; 1.0009x over previous
import functools

import jax
import jax.numpy as jnp
import numpy as np
from jax import lax
from jax.experimental import pallas as pl
from jax.experimental.pallas import tpu as pltpu
from jax.experimental.pallas import tpu_sc as plsc

GRID = 32
STEP = 1.0 / GRID
NUM_STEPS = 4
HEADS = 4
HID = 64
B = 64
N = B * GRID
F1 = HEADS * HID


_HI = jax.lax.Precision.HIGHEST


def _dotx(a, b):
    return jnp.dot(a, b, preferred_element_type=jnp.float32, precision=_HI)


def _b32(x):
    return x.astype(jnp.bfloat16).astype(jnp.float32)

def _roll(h, s):
    if s == 0:
        return h
    return jnp.concatenate([h[:, -s:, :], h[:, :-s, :]], axis=1)


def _elu(x):
    return jnp.where(x > 0, x, jnp.exp(jnp.minimum(x, 0.0)) - 1.0)


def _leaky(x):
    return jnp.where(x >= 0, x, 0.2 * x)


def _src_3d(ts):
    xi = -(ts + 2 * ts + 2 * ts + ts) / 6.0
    gio = lax.broadcasted_iota(jnp.int32, (B, GRID, 1), 1).astype(jnp.float32)
    x_n = (gio + 0.5) * STEP + xi
    x_n = x_n - jnp.floor(x_n)
    kio = lax.broadcasted_iota(jnp.int32, (B, GRID, GRID), 2).astype(jnp.float32)
    cmp = ((kio + 0.5) * STEP <= x_n).astype(jnp.int32)
    indx = jnp.sum(cmp, axis=2, keepdims=True) - 1
    return indx & 31, (indx + 1) & 31, xi


def _idx_body(ts_ref, f1_ref, f2_ref):
    src1, src2, _ = _src_3d(ts_ref[...])
    bio = lax.broadcasted_iota(jnp.int32, (B, GRID, 1), 0)
    f1_ref[...] = src1 + bio * GRID
    f2_ref[...] = src2 + bio * GRID


def _tc1_body(ts_ref, xg_ref, *refs):
    conv = [(refs[2 * i][...], refs[2 * i + 1][...]) for i in range(6)]
    wl, bl, wr, br = (refs[12][...], refs[13][...], refs[14][...], refs[15][...])
    xl_ref, xr_ref = refs[16], refs[17]
    ts = ts_ref[...]
    xi = -(ts + 2 * ts + 2 * ts + ts) / 6.0
    xi_ch = jnp.broadcast_to(xi * np.float32(GRID), (B, GRID, 1))
    h = jnp.concatenate([xg_ref[...], xi_ch], axis=2)
    for (W5, b5) in conv:
        ci5, co = W5.shape
        X5 = jnp.concatenate([_roll(h, s) for s in (2, 1, 0, -1, -2)], axis=2)
        hh = jnp.dot(X5.reshape(N, ci5), W5, preferred_element_type=jnp.float32) + b5
        h = _elu(hh).reshape(B, GRID, co)
    xc2 = h.reshape(N, 128)
    xl_ref[...] = jnp.dot(xc2, wl, preferred_element_type=jnp.float32) + bl
    xr_ref[...] = jnp.dot(xc2, wr, preferred_element_type=jnp.float32) + br


def _att_lin_body(a1_ref, a2_ref, xl_ref, xr_ref, am_ref, hmt_ref, bs_ref,
                  wl2_ref, bl2_ref, wr2_ref, br2_ref, xl2_ref, xr2_ref):
    A1, A2 = a1_ref[...], a2_ref[...]
    xl, xr = xl_ref[...], xr_ref[...]
    am, hmt, bs = am_ref[...], hmt_ref[...], bs_ref[...]
    l1 = _dotx(_leaky(A1 + xr), am)
    l2 = _dotx(_leaky(A2 + xr), am)
    ls = _dotx(_leaky(xl + xr), am)
    m = jnp.maximum(jnp.maximum(l1, l2), ls)
    a1 = jnp.exp(l1 - m)
    a2 = jnp.exp(l2 - m)
    as_ = jnp.exp(ls - m)
    den = a1 + a2 + as_
    wb = lambda a: _dotx(a / den, hmt)
    xc2 = wb(a1) * A1 + wb(a2) * A2 + wb(as_) * xl + bs
    xl2_ref[...] = jnp.dot(xc2, wl2_ref[...], preferred_element_type=jnp.float32) + bl2_ref[...]
    xr2_ref[...] = jnp.dot(xc2, wr2_ref[...], preferred_element_type=jnp.float32) + br2_ref[...]


def _fin_body(ts_ref, a1_ref, a2_ref, xl_ref, xr_ref, am_ref, hmt_ref,
              bs_ref, w2_ref, xg_ref, xlane_ref,
              newxg_ref, newxlane_ref, pred_ref):
    A1, A2 = a1_ref[...], a2_ref[...]
    xl, xr = xl_ref[...], xr_ref[...]
    am, hmt, bs = am_ref[...], hmt_ref[...], bs_ref[...]
    l1 = _dotx(_leaky(A1 + xr), am)
    l2 = _dotx(_leaky(A2 + xr), am)
    ls = _dotx(_leaky(xl + xr), am)
    m = jnp.maximum(jnp.maximum(l1, l2), ls)
    a1 = jnp.exp(l1 - m)
    a2 = jnp.exp(l2 - m)
    as_ = jnp.exp(ls - m)
    den = a1 + a2 + as_
    wb = lambda a: _dotx(a / den, hmt)
    xc2 = _elu(wb(a1) * A1 + wb(a2) * A2 + wb(as_) * xl + bs)

    src1, src2, _ = _src_3d(ts_ref[...])
    sio = lax.broadcasted_iota(jnp.int32, (B, GRID, GRID), 2)
    oh1 = (src1 == sio).astype(jnp.float32)
    oh2 = (src2 == sio).astype(jnp.float32)
    ohsum = oh1 + oh2
    deg = jnp.sum(ohsum, axis=1, keepdims=True)
    gio_i = lax.broadcasted_iota(jnp.int32, (B, GRID, GRID), 1)
    ident = (gio_i == sio).astype(jnp.float32)

    xc3 = xc2.reshape(B, GRID, F1)
    v3 = jnp.sum(_b32(xc3) * _b32(w2_ref[...]), axis=2, keepdims=True)
    Sv = jnp.sum(ohsum * v3, axis=1, keepdims=True)
    q = (Sv - 1.0) / deg
    q1 = jnp.sum(oh1 * q, axis=2, keepdims=True)
    q2 = jnp.sum(oh2 * q, axis=2, keepdims=True)
    xlane = xlane_ref[...]
    xs1 = jnp.sum(oh1 * xlane, axis=2, keepdims=True)
    xs2 = jnp.sum(oh2 * xlane, axis=2, keepdims=True)
    xg3 = xs1 * (v3 - q1) + xs2 * (v3 - q2)
    newxg_ref[...] = xg3
    newxlane_ref[...] = jnp.sum(ident * xg3, axis=1, keepdims=True)
    pred_ref[...] = xg3.reshape(N, 1)


def _fin_conv_body(ts_ref, a1_ref, a2_ref, xl_ref, xr_ref, am_ref, hmt_ref,
                   bs_ref, w2_ref, xg_ref, xlane_ref, *refs):
    conv = [(refs[2 * i][...], refs[2 * i + 1][...]) for i in range(6)]
    wl, bl, wr, br = (refs[12][...], refs[13][...], refs[14][...], refs[15][...])
    pred_ref, newxlane_ref, xl1_ref, xr1_ref = refs[16], refs[17], refs[18], refs[19]

    A1, A2 = a1_ref[...], a2_ref[...]
    xl, xr = xl_ref[...], xr_ref[...]
    am, hmt, bs = am_ref[...], hmt_ref[...], bs_ref[...]
    l1 = _dotx(_leaky(A1 + xr), am)
    l2 = _dotx(_leaky(A2 + xr), am)
    ls = _dotx(_leaky(xl + xr), am)
    m = jnp.maximum(jnp.maximum(l1, l2), ls)
    a1 = jnp.exp(l1 - m)
    a2 = jnp.exp(l2 - m)
    as_ = jnp.exp(ls - m)
    den = a1 + a2 + as_
    wb = lambda a: _dotx(a / den, hmt)
    xc2 = _elu(wb(a1) * A1 + wb(a2) * A2 + wb(as_) * xl + bs)

    src1, src2, xi = _src_3d(ts_ref[...])
    sio = lax.broadcasted_iota(jnp.int32, (B, GRID, GRID), 2)
    oh1 = (src1 == sio).astype(jnp.float32)
    oh2 = (src2 == sio).astype(jnp.float32)
    ohsum = oh1 + oh2
    deg = jnp.sum(ohsum, axis=1, keepdims=True)
    gio_i = lax.broadcasted_iota(jnp.int32, (B, GRID, GRID), 1)
    ident = (gio_i == sio).astype(jnp.float32)

    xc3 = xc2.reshape(B, GRID, F1)
    v3 = jnp.sum(_b32(xc3) * _b32(w2_ref[...]), axis=2, keepdims=True)
    Sv = jnp.sum(ohsum * v3, axis=1, keepdims=True)
    q = (Sv - 1.0) / deg
    q1 = jnp.sum(oh1 * q, axis=2, keepdims=True)
    q2 = jnp.sum(oh2 * q, axis=2, keepdims=True)
    xlane = xlane_ref[...]
    xs1 = jnp.sum(oh1 * xlane, axis=2, keepdims=True)
    xs2 = jnp.sum(oh2 * xlane, axis=2, keepdims=True)
    xg3 = xs1 * (v3 - q1) + xs2 * (v3 - q2)
    pred_ref[...] = xg3.reshape(N, 1)
    newxlane_ref[...] = jnp.sum(ident * xg3, axis=1, keepdims=True)

    xi_ch = jnp.broadcast_to(xi * np.float32(GRID), (B, GRID, 1))
    h = jnp.concatenate([xg3, xi_ch], axis=2)
    for (W5, b5) in conv:
        ci5, co = W5.shape
        X5 = jnp.concatenate([_roll(h, s) for s in (2, 1, 0, -1, -2)], axis=2)
        hh = jnp.dot(X5.reshape(N, ci5), W5, preferred_element_type=jnp.float32) + b5
        h = _elu(hh).reshape(B, GRID, co)
    xc = h.reshape(N, 128)
    xl1_ref[...] = jnp.dot(xc, wl, preferred_element_type=jnp.float32) + bl
    xr1_ref[...] = jnp.dot(xc, wr, preferred_element_type=jnp.float32) + br


_info = None


def _sc_gather(table, idx1, idx2):
    NW = 32
    b_per_w = N // NW
    mesh = plsc.VectorSubcoreMesh(core_axis_name="c", subcore_axis_name="s")

    @functools.partial(
        pl.kernel, mesh=mesh,
        out_type=(jax.ShapeDtypeStruct((N, F1), jnp.float32),
                  jax.ShapeDtypeStruct((N, F1), jnp.float32)),
        scratch_types=[
            pltpu.VMEM((b_per_w,), jnp.int32),
            pltpu.VMEM((b_per_w,), jnp.int32),
            pltpu.VMEM((b_per_w, F1), jnp.float32),
            pltpu.VMEM((b_per_w, F1), jnp.float32),
            pltpu.SemaphoreType.DMA,
            pltpu.SemaphoreType.DMA,
        ],
    )
    def k(table_hbm, i1_hbm, i2_hbm, o1_hbm, o2_hbm,
          i1_v, i2_v, r1_v, r2_v, sem1, sem2):
        wid = lax.axis_index("s") * 2 + lax.axis_index("c")
        base = wid * b_per_w
        pltpu.sync_copy(i1_hbm.at[pl.ds(base, b_per_w)], i1_v)
        pltpu.sync_copy(i2_hbm.at[pl.ds(base, b_per_w)], i2_v)
        c1 = pltpu.async_copy(table_hbm.at[i1_v], r1_v, sem1)
        c2 = pltpu.async_copy(table_hbm.at[i2_v], r2_v, sem2)
        c1.wait()
        c2.wait()
        pltpu.sync_copy(r1_v, o1_hbm.at[pl.ds(base, b_per_w)])
        pltpu.sync_copy(r2_v, o2_hbm.at[pl.ds(base, b_per_w)])

    return k(table, idx1, idx2)


def _tc_call(body, args, out_shapes):
    return pl.pallas_call(
        body,
        out_shape=out_shapes,
    )(*args)


@jax.jit
def kernel(x, cur_time, time_step, conv_params, gat1, gat2, lin_W, lin_b):
    del cur_time, lin_b
    ts3 = time_step.astype(jnp.float32).reshape(B, 1, 1)
    xg3 = x.reshape(B, GRID, 1)
    xlane = x.reshape(B, 1, GRID)

    flat_conv = []
    for (W, b) in conv_params:
        co, ci, _ = W.shape
        flat_conv.append(jnp.transpose(W, (2, 1, 0)).reshape(5 * ci, co))
        flat_conv.append(b.reshape(1, co))

    def gat_args(g):
        Wl, bl, Wr, br, att, bias = g
        attf = att.reshape(F1)
        hsel = (np.arange(F1)[:, None] // HID) == np.arange(HEADS)[None, :]
        am = attf[:, None] * jnp.asarray(hsel, dtype=jnp.float32)
        return [Wl, bl.reshape(1, F1), Wr, br.reshape(1, F1), am,
                bias.reshape(1, F1)]

    g1 = gat_args(gat1)
    g2 = gat_args(gat2)
    hmt = jnp.asarray((np.arange(F1)[None, :] // HID)
                      == np.arange(HEADS)[:, None], dtype=jnp.float32)
    w2 = lin_W[F1:, 0].reshape(1, 1, F1)

    S = jax.ShapeDtypeStruct
    f1g, f2g = _tc_call(_idx_body, [ts3],
                        (S((B, GRID, 1), jnp.int32), S((B, GRID, 1), jnp.int32)))
    idx1 = f1g.reshape(N)
    idx2 = f2g.reshape(N)

    preds = []
    xl1, xr1 = _tc_call(
        _tc1_body,
        [ts3, xg3] + flat_conv + [g1[0], g1[1], g1[2], g1[3]],
        (S((N, F1), jnp.float32), S((N, F1), jnp.float32)))
    for step in range(NUM_STEPS):
        A1, A2 = _sc_gather(xl1, idx1, idx2)
        xl2, xr2 = _tc_call(
            _att_lin_body,
            [A1, A2, xl1, xr1, g1[4], hmt, g1[5], g2[0], g2[1], g2[2], g2[3]],
            (S((N, F1), jnp.float32), S((N, F1), jnp.float32)))
        B1, B2 = _sc_gather(xl2, idx1, idx2)
        if step < NUM_STEPS - 1:
            pred, xlane, xl1, xr1 = _tc_call(
                _fin_conv_body,
                [ts3, B1, B2, xl2, xr2, g2[4], hmt, g2[5], w2, xg3, xlane]
                + flat_conv + [g1[0], g1[1], g1[2], g1[3]],
                (S((N, 1), jnp.float32), S((B, 1, GRID), jnp.float32),
                 S((N, F1), jnp.float32), S((N, F1), jnp.float32)))
        else:
            _, xlane, pred = _tc_call(
                _fin_body,
                [ts3, B1, B2, xl2, xr2, g2[4], hmt, g2[5], w2, xg3, xlane],
                (S((B, GRID, 1), jnp.float32), S((B, 1, GRID), jnp.float32),
                 S((N, 1), jnp.float32)))
        preds.append(pred)
    return jnp.concatenate(preds, axis=1)
